# baseline (speedup 1.0000x reference)
import functools

import jax
import jax.numpy as jnp
from jax import lax
from jax.experimental import pallas as pl
from jax.experimental.pallas import tpu as pltpu
from jax.experimental.pallas import tpu_sc as plsc

BATCH = 16384
EMBED = 64
NC = 2
NS = 16
LANES = 16
NW = NC * NS
BPW = BATCH // NW
CHUNK = 128
NCHUNK = BPW // CHUNK


def _mf_body(uid_hbm, sid_hbm, utab_hbm, stab_hbm, out_hbm,
             uid_v, sid_v, urows, srows, outv, sem_idx, sem_u, sem_s):
    wid = lax.axis_index("s") * NC + lax.axis_index("c")
    base = wid * BPW

    idx_copies = []
    for i in range(NCHUNK):
        idx_copies.append(pltpu.async_copy(
            uid_hbm.at[pl.ds(base + i * CHUNK, CHUNK)], uid_v.at[i], sem_idx))
        idx_copies.append(pltpu.async_copy(
            sid_hbm.at[pl.ds(base + i * CHUNK, CHUNK)], sid_v.at[i], sem_idx))
    for c in idx_copies:
        c.wait()

    u_copies = [pltpu.async_copy(utab_hbm.at[uid_v.at[i]],
                                 urows.at[pl.ds(i * CHUNK, CHUNK)], sem_u)
                for i in range(NCHUNK)]
    s_copies = [pltpu.async_copy(stab_hbm.at[sid_v.at[i]],
                                 srows.at[pl.ds(i * CHUNK, CHUNK)], sem_s)
                for i in range(NCHUNK)]
    for c in u_copies:
        c.wait()
    for c in s_copies:
        c.wait()

    lane = lax.iota(jnp.int32, LANES)
    def group(t, _):
        rows = t * LANES + lane
        acc0 = jnp.zeros((LANES,), jnp.float32)
        acc1 = jnp.zeros((LANES,), jnp.float32)
        for j in range(0, EMBED, 2):
            cj0 = jnp.full((LANES,), j, jnp.int32)
            cj1 = jnp.full((LANES,), j + 1, jnp.int32)
            u0 = plsc.load_gather(urows, [rows, cj0])
            s0 = plsc.load_gather(srows, [rows, cj0])
            u1 = plsc.load_gather(urows, [rows, cj1])
            s1 = plsc.load_gather(srows, [rows, cj1])
            acc0 = acc0 + u0 * s0
            acc1 = acc1 + u1 * s1
        dot = acc0 + acc1
        rating = 10.0 / (1.0 + jnp.exp(-dot))
        outv[pl.ds(t * LANES, LANES)] = rating
        return _

    lax.fori_loop(0, BPW // LANES, group, None)

    pltpu.sync_copy(outv, out_hbm.at[pl.ds(base, BPW)])


@functools.partial(jax.jit, static_argnums=())
def kernel(user_id, song_id, user_embedding, song_embedding):
    mesh = plsc.VectorSubcoreMesh(core_axis_name="c", subcore_axis_name="s")
    k = pl.kernel(
        _mf_body,
        mesh=mesh,
        compiler_params=pltpu.CompilerParams(
            needs_layout_passes=False, use_tc_tiling_on_sc=False),
        out_type=jax.ShapeDtypeStruct((BATCH,), jnp.float32),
        scratch_types=[
            pltpu.VMEM((NCHUNK, CHUNK), jnp.int32),
            pltpu.VMEM((NCHUNK, CHUNK), jnp.int32),
            pltpu.VMEM((BPW, EMBED), jnp.float32),
            pltpu.VMEM((BPW, EMBED), jnp.float32),
            pltpu.VMEM((BPW,), jnp.float32),
            pltpu.SemaphoreType.DMA,
            pltpu.SemaphoreType.DMA,
            pltpu.SemaphoreType.DMA,
        ],
    )
    return k(user_id.astype(jnp.int32), song_id.astype(jnp.int32),
             user_embedding, song_embedding)


# speedup vs baseline: 1.5992x; 1.5992x over previous
import jax
import jax.numpy as jnp
from jax import lax
from jax.experimental import pallas as pl
from jax.experimental.pallas import tpu as pltpu
from jax.experimental.pallas import tpu_sc as plsc

BATCH = 16384
EMBED = 64
NC = 2
NS = 16
LANES = 16
NW = NC * NS
BPW = BATCH // NW
CHUNK = 128
NCHUNK = BPW // CHUNK


def _mf_body(uid_hbm, sid_hbm, utab_hbm, stab_hbm, out_hbm,
             uids_v, sids_v,
             ubuf0, ubuf1, sbuf0, sbuf1, outv,
             sem_u0, sem_u1, sem_s0, sem_s1):
    wid = lax.axis_index("s") * NC + lax.axis_index("c")
    base = wid * BPW

    pltpu.sync_copy(uid_hbm.at[pl.ds(base, BPW)], uids_v)
    pltpu.sync_copy(sid_hbm.at[pl.ds(base, BPW)], sids_v)

    ubufs = (ubuf0, ubuf1)
    sbufs = (sbuf0, sbuf1)
    usems = (sem_u0, sem_u1)
    ssems = (sem_s0, sem_s1)

    def fire(ci):
        ub, sb = ubufs[ci % 2], sbufs[ci % 2]
        us, ss = usems[ci % 2], ssems[ci % 2]

        def fire_16(t, _):
            uvec = uids_v[pl.ds(ci * CHUNK + t * LANES, LANES)]
            svec = sids_v[pl.ds(ci * CHUNK + t * LANES, LANES)]
            for j in range(LANES):
                uid = uvec[j]
                sid = svec[j]
                pltpu.async_copy(utab_hbm.at[pl.ds(uid, 1), :],
                                 ub.at[pl.ds(t * LANES + j, 1), :], us)
                pltpu.async_copy(stab_hbm.at[pl.ds(sid, 1), :],
                                 sb.at[pl.ds(t * LANES + j, 1), :], ss)
            return _

        lax.fori_loop(0, CHUNK // LANES, fire_16, None)

    def drain(ci):
        ub, sb = ubufs[ci % 2], sbufs[ci % 2]
        us, ss = usems[ci % 2], ssems[ci % 2]

        def drain_one(k, _):
            pltpu.make_async_copy(utab_hbm.at[pl.ds(0, 1), :],
                                  ub.at[pl.ds(0, 1), :], us).wait()
            pltpu.make_async_copy(stab_hbm.at[pl.ds(0, 1), :],
                                  sb.at[pl.ds(0, 1), :], ss).wait()
            return _

        lax.fori_loop(0, CHUNK, drain_one, None)

    lane = lax.iota(jnp.int32, LANES)

    def compute(ci):
        ub, sb = ubufs[ci % 2], sbufs[ci % 2]

        def group(t, _):
            rows = t * LANES + lane
            acc0 = jnp.zeros((LANES,), jnp.float32)
            acc1 = jnp.zeros((LANES,), jnp.float32)
            for j in range(0, EMBED, 2):
                cj0 = jnp.full((LANES,), j, jnp.int32)
                cj1 = jnp.full((LANES,), j + 1, jnp.int32)
                u0 = plsc.load_gather(ub, [rows, cj0])
                s0 = plsc.load_gather(sb, [rows, cj0])
                u1 = plsc.load_gather(ub, [rows, cj1])
                s1 = plsc.load_gather(sb, [rows, cj1])
                acc0 = acc0 + u0 * s0
                acc1 = acc1 + u1 * s1
            dot = acc0 + acc1
            rating = 10.0 / (1.0 + jnp.exp(-dot))
            outv[pl.ds(ci * CHUNK + t * LANES, LANES)] = rating
            return _

        lax.fori_loop(0, CHUNK // LANES, group, None)

    fire(0)
    for ci in range(NCHUNK):
        if ci + 1 < NCHUNK:
            fire(ci + 1)
        drain(ci)
        compute(ci)

    pltpu.sync_copy(outv, out_hbm.at[pl.ds(base, BPW)])


def kernel(user_id, song_id, user_embedding, song_embedding):
    mesh = plsc.VectorSubcoreMesh(core_axis_name="c", subcore_axis_name="s")
    k = pl.kernel(
        _mf_body,
        mesh=mesh,
        compiler_params=pltpu.CompilerParams(
            needs_layout_passes=False, use_tc_tiling_on_sc=True),
        out_type=jax.ShapeDtypeStruct((BATCH,), jnp.float32),
        scratch_types=[
            pltpu.VMEM((BPW,), jnp.int32),
            pltpu.VMEM((BPW,), jnp.int32),
            pltpu.VMEM((CHUNK, EMBED), jnp.float32),
            pltpu.VMEM((CHUNK, EMBED), jnp.float32),
            pltpu.VMEM((CHUNK, EMBED), jnp.float32),
            pltpu.VMEM((CHUNK, EMBED), jnp.float32),
            pltpu.VMEM((BPW,), jnp.float32),
            pltpu.SemaphoreType.DMA,
            pltpu.SemaphoreType.DMA,
            pltpu.SemaphoreType.DMA,
            pltpu.SemaphoreType.DMA,
        ],
    )
    return k(user_id.astype(jnp.int32), song_id.astype(jnp.int32),
             user_embedding, song_embedding)
